# baseline (speedup 1.0000x reference)
import functools
import jax
import jax.numpy as jnp
from jax import lax
from jax.experimental import pallas as pl
from jax.experimental.pallas import tpu as pltpu
from jax.experimental.pallas import tpu_sc as plsc

N = 10000
E = 160000
IN = 16
HID = 16
OUT = 16
ED = 16
H = 128

NC = 2
NS = 16
NW = NC * NS
CH = 128
NCHT = E // CH
NHALF = 2
NCHH = NCHT // NHALF
EH = NCHH * CH
RPS = 632
NP = NS * RPS

BE = 2000

_MESH = plsc.VectorSubcoreMesh(core_axis_name="c", subcore_axis_name="s")
_SC_PARAMS = pltpu.CompilerParams(use_tc_tiling_on_sc=False)


def _make_gather(chunk0):
    nchw = NCHH // NW
    nxtra = NCHH - nchw * NW

    @functools.partial(
        pl.kernel,
        out_type=jax.ShapeDtypeStruct((EH, 16), jnp.float32),
        mesh=_MESH,
        scratch_types=[
            pltpu.VMEM((nchw + 1, CH), jnp.int32),
            pltpu.VMEM(((nchw + 1) * CH, 16), jnp.float32),
            pltpu.SemaphoreType.DMA,
        ],
        compiler_params=_SC_PARAMS,
    )
    def gather(tab_hbm, idx_hbm, out_hbm, idx_v, rows_v, sem):
        c = lax.axis_index("c")
        s = lax.axis_index("s")
        wid = s * NC + c
        extra = jnp.where(wid < nxtra, 1, 0)
        crel = wid * nchw + jnp.minimum(wid, nxtra)
        cabs = chunk0 + crel
        nch = nchw + extra
        pltpu.sync_copy(idx_hbm.at[pl.ds(cabs, nchw)],
                        idx_v.at[pl.ds(0, nchw)])

        @pl.when(extra == 1)
        def _():
            pltpu.sync_copy(idx_hbm.at[pl.ds(cabs + nchw, 1)],
                            idx_v.at[pl.ds(nchw, 1)])

        def fire(j, carry):
            pltpu.async_copy(tab_hbm.at[idx_v.at[j]],
                             rows_v.at[pl.ds(j * CH, CH)], sem)
            return carry

        lax.fori_loop(0, nch, fire, 0)

        def drain(j, carry):
            pltpu.make_async_copy(tab_hbm.at[idx_v.at[0]],
                                  rows_v.at[pl.ds(0, CH)], sem).wait()
            return carry

        lax.fori_loop(0, nch, drain, 0)
        pltpu.sync_copy(rows_v.at[pl.ds(0, nchw * CH)],
                        out_hbm.at[pl.ds(crel * CH, nchw * CH)])

        @pl.when(extra == 1)
        def _():
            pltpu.sync_copy(rows_v.at[pl.ds(nchw * CH, CH)],
                            out_hbm.at[pl.ds((crel + nchw) * CH, CH)])

    return gather


def _make_scatter(chunk0, with_cnt):
    nchw = NCHH // NW
    nxtra = NCHH - nchw * NW
    out_t = jax.ShapeDtypeStruct((NC * NP, 16), jnp.float32)
    scratch = [
        pltpu.VMEM((nchw + 1, CH), jnp.int32),
        pltpu.VMEM(((nchw + 1) * CH, 16), jnp.float32),
        pltpu.VMEM_SHARED((NP, 16), jnp.float32),
        pltpu.SemaphoreType.DMA,
    ]
    if with_cnt:
        out_t = (out_t, jax.ShapeDtypeStruct((NC * NP, 16), jnp.float32))
        scratch = scratch[:2] + [
            pltpu.VMEM((CH, 16), jnp.float32),
            pltpu.VMEM_SHARED((NP, 16), jnp.float32),
        ] + scratch[2:]

    def body(idx_hbm, vals_hbm, zeros_hbm, *rest):
        if with_cnt:
            (ones_hbm, out_hbm, cout_hbm, idx_v, vals_v, ones_v, cacc_sh,
             acc_sh, sem) = rest
        else:
            (out_hbm, idx_v, vals_v, acc_sh, sem) = rest
        c = lax.axis_index("c")
        s = lax.axis_index("s")
        wid = s * NC + c
        extra = jnp.where(wid < nxtra, 1, 0)
        crel = wid * nchw + jnp.minimum(wid, nxtra)
        cabs = chunk0 + crel
        nch = nchw + extra
        pltpu.sync_copy(zeros_hbm.at[pl.ds(s * RPS, RPS)],
                        acc_sh.at[pl.ds(s * RPS, RPS)])
        if with_cnt:
            pltpu.sync_copy(zeros_hbm.at[pl.ds(s * RPS, RPS)],
                            cacc_sh.at[pl.ds(s * RPS, RPS)])
            pltpu.sync_copy(ones_hbm, ones_v)
        pltpu.sync_copy(idx_hbm.at[pl.ds(cabs, nchw)],
                        idx_v.at[pl.ds(0, nchw)])
        pltpu.sync_copy(vals_hbm.at[pl.ds(crel * CH, nchw * CH)],
                        vals_v.at[pl.ds(0, nchw * CH)])

        @pl.when(extra == 1)
        def _():
            pltpu.sync_copy(idx_hbm.at[pl.ds(cabs + nchw, 1)],
                            idx_v.at[pl.ds(nchw, 1)])
            pltpu.sync_copy(vals_hbm.at[pl.ds((crel + nchw) * CH, CH)],
                            vals_v.at[pl.ds(nchw * CH, CH)])

        plsc.subcore_barrier()

        def fire(j, carry):
            pltpu.async_copy(vals_v.at[pl.ds(j * CH, CH)],
                             acc_sh.at[idx_v.at[j]], sem, add=True)
            if with_cnt:
                pltpu.async_copy(ones_v, cacc_sh.at[idx_v.at[j]], sem,
                                 add=True)
            return carry

        lax.fori_loop(0, nch, fire, 0)
        n_waits = nch * 2 if with_cnt else nch

        def drain(j, carry):
            pltpu.make_async_copy(vals_v.at[pl.ds(0, CH)],
                                  acc_sh.at[idx_v.at[0]], sem).wait()
            return carry

        lax.fori_loop(0, n_waits, drain, 0)
        plsc.subcore_barrier()
        pltpu.sync_copy(acc_sh.at[pl.ds(s * RPS, RPS)],
                        out_hbm.at[pl.ds(c * NP + s * RPS, RPS)])
        if with_cnt:
            pltpu.sync_copy(cacc_sh.at[pl.ds(s * RPS, RPS)],
                            cout_hbm.at[pl.ds(c * NP + s * RPS, RPS)])

    return functools.partial(
        pl.kernel, mesh=_MESH, compiler_params=_SC_PARAMS,
        out_type=out_t, scratch_types=scratch)(body)


_gather_h = tuple(_make_gather(h * NCHH) for h in range(NHALF))
_scatter_cnt_h = tuple(_make_scatter(h * NCHH, True) for h in range(NHALF))
_scatter_h = tuple(_make_scatter(h * NCHH, False) for h in range(NHALF))


def _msg_body(ea_ref, xs_ref, w1_ref, b1_ref, w2_ref, b2_ref, p_ref, q_ref,
              out_ref):
    ea = ea_ref[...]
    xs = xs_ref[...]
    h = jnp.maximum(
        jnp.dot(ea, w1_ref[...], preferred_element_type=jnp.float32)
        + b1_ref[...], 0.0)
    wf = jnp.dot(h, w2_ref[...], preferred_element_type=jnp.float32) \
        + b2_ref[...]
    xr = jnp.dot(xs, p_ref[...], preferred_element_type=jnp.float32)
    out_ref[...] = jnp.dot(xr * wf, q_ref[...],
                           preferred_element_type=jnp.float32)


def _edge_messages(half, ea, xs, w1, b1, w2, b2, ic, oc):
    grid = EH // BE
    off = half * grid
    ii = jnp.arange(ic * oc) // oc
    oo = jnp.arange(ic * oc) % oc
    p_mat = (ii[None, :] == jnp.arange(ic)[:, None]).astype(jnp.float32)
    q_mat = (oo[:, None] == jnp.arange(oc)[None, :]).astype(jnp.float32)
    b1r = b1.reshape(1, -1)
    b2r = b2.reshape(1, -1)

    def fixed(a):
        return pl.BlockSpec(a.shape, lambda i: (0,) * a.ndim)

    return pl.pallas_call(
        _msg_body,
        grid=(grid,),
        in_specs=[
            pl.BlockSpec((BE, ic), lambda i: (i + off, 0)),
            pl.BlockSpec((BE, ic), lambda i: (i, 0)),
            fixed(w1), fixed(b1r), fixed(w2), fixed(b2r),
            fixed(p_mat), fixed(q_mat),
        ],
        out_specs=pl.BlockSpec((BE, oc), lambda i: (i, 0)),
        out_shape=jax.ShapeDtypeStruct((EH, oc), jnp.float32),
    )(ea, xs, w1, b1r, w2, b2r, p_mat, q_mat)


def _node_body(x_ref, sa_ref, sb_ref, ca_ref, cb_ref, root_ref, bias_ref,
               out_ref, *, do_relu):
    s = (sa_ref[0:N] + sa_ref[NP:NP + N]
         + sb_ref[0:N] + sb_ref[NP:NP + N])
    cnt = (ca_ref[0:N] + ca_ref[NP:NP + N]
           + cb_ref[0:N] + cb_ref[NP:NP + N])
    inv = 1.0 / jnp.maximum(cnt, 1.0)
    r = jnp.dot(x_ref[...], root_ref[...],
                preferred_element_type=jnp.float32)
    o = r + s * inv + bias_ref[...]
    if do_relu:
        o = jnp.maximum(o, 0.0)
    out_ref[...] = o


def _node_combine(x, sa, sb, ca, cb, root, bias, do_relu):
    return pl.pallas_call(
        functools.partial(_node_body, do_relu=do_relu),
        out_shape=jax.ShapeDtypeStruct((N, 16), jnp.float32),
    )(x, sa, sb, ca, cb, root, bias.reshape(1, -1))


def kernel(x, edge_index, edge_attr, nn1_w1, nn1_b1, nn1_w2, nn1_b2, root1,
           bias1, nn2_w1, nn2_b1, nn2_w2, nn2_b2, root2, bias2):
    src2 = edge_index[0].astype(jnp.int32).reshape(NCHT, CH)
    dst2 = edge_index[1].astype(jnp.int32).reshape(NCHT, CH)
    zeros_np = jnp.zeros((NP, 16), jnp.float32)
    ones_ch = jnp.ones((CH, 16), jnp.float32)

    xa = _gather_h[0](x, src2)
    xb = _gather_h[1](x, src2)
    m1a = _edge_messages(0, edge_attr, xa, nn1_w1, nn1_b1, nn1_w2, nn1_b2,
                         IN, HID)
    s1a, c1a = _scatter_cnt_h[0](dst2, m1a, zeros_np, ones_ch)
    m1b = _edge_messages(1, edge_attr, xb, nn1_w1, nn1_b1, nn1_w2, nn1_b2,
                         IN, HID)
    s1b, c1b = _scatter_cnt_h[1](dst2, m1b, zeros_np, ones_ch)
    h = _node_combine(x, s1a, s1b, c1a, c1b, root1, bias1, True)

    ha = _gather_h[0](h, src2)
    hb = _gather_h[1](h, src2)
    m2a = _edge_messages(0, edge_attr, ha, nn2_w1, nn2_b1, nn2_w2, nn2_b2,
                         HID, OUT)
    s2a = _scatter_h[0](dst2, m2a, zeros_np)
    m2b = _edge_messages(1, edge_attr, hb, nn2_w1, nn2_b1, nn2_w2, nn2_b2,
                         HID, OUT)
    s2b = _scatter_h[1](dst2, m2b, zeros_np)
    out = _node_combine(h, s2a, s2b, c1a, c1b, root2, bias2, False)
    return out


# speedup vs baseline: 1.9933x; 1.9933x over previous
import functools
import jax
import jax.numpy as jnp
from jax import lax
from jax.experimental import pallas as pl
from jax.experimental.pallas import tpu as pltpu
from jax.experimental.pallas import tpu_sc as plsc

N = 10000
E = 160000
IN = 16
HID = 16
OUT = 16
ED = 16
H = 128

NC = 2
NS = 16
NW = NC * NS
CH = 128
NCHT = E // CH
NHALF = 2
NCHH = NCHT // NHALF
EH = NCHH * CH
RPS = 632
NP = NS * RPS

BE = 3200
BR = BE // 8

_MESH = plsc.VectorSubcoreMesh(core_axis_name="c", subcore_axis_name="s")
_SC_PARAMS = pltpu.CompilerParams(use_tc_tiling_on_sc=False)


def _make_gather(chunk0):
    nchw = NCHH // NW
    nxtra = NCHH - nchw * NW

    @functools.partial(
        pl.kernel,
        out_type=jax.ShapeDtypeStruct((EH, 16), jnp.float32),
        mesh=_MESH,
        scratch_types=[
            pltpu.VMEM((nchw + 1, CH), jnp.int32),
            pltpu.VMEM(((nchw + 1) * CH, 16), jnp.float32),
            pltpu.SemaphoreType.DMA,
        ],
        compiler_params=_SC_PARAMS,
    )
    def gather(tab_hbm, idx_hbm, out_hbm, idx_v, rows_v, sem):
        c = lax.axis_index("c")
        s = lax.axis_index("s")
        wid = s * NC + c
        extra = jnp.where(wid < nxtra, 1, 0)
        crel = wid * nchw + jnp.minimum(wid, nxtra)
        cabs = chunk0 + crel
        nch = nchw + extra
        pltpu.sync_copy(idx_hbm.at[pl.ds(cabs, nchw)],
                        idx_v.at[pl.ds(0, nchw)])

        @pl.when(extra == 1)
        def _():
            pltpu.sync_copy(idx_hbm.at[pl.ds(cabs + nchw, 1)],
                            idx_v.at[pl.ds(nchw, 1)])

        def fire(j, carry):
            pltpu.async_copy(tab_hbm.at[idx_v.at[j]],
                             rows_v.at[pl.ds(j * CH, CH)], sem)
            return carry

        lax.fori_loop(0, nch, fire, 0)

        def drain(j, carry):
            pltpu.make_async_copy(tab_hbm.at[idx_v.at[0]],
                                  rows_v.at[pl.ds(0, CH)], sem).wait()
            return carry

        lax.fori_loop(0, nch, drain, 0)
        pltpu.sync_copy(rows_v.at[pl.ds(0, nchw * CH)],
                        out_hbm.at[pl.ds(crel * CH, nchw * CH)])

        @pl.when(extra == 1)
        def _():
            pltpu.sync_copy(rows_v.at[pl.ds(nchw * CH, CH)],
                            out_hbm.at[pl.ds((crel + nchw) * CH, CH)])

    return gather


def _make_scatter(chunk0, with_cnt):
    nchw = NCHH // NW
    nxtra = NCHH - nchw * NW
    out_t = jax.ShapeDtypeStruct((NC * NP, 16), jnp.float32)
    scratch = [
        pltpu.VMEM((nchw + 1, CH), jnp.int32),
        pltpu.VMEM(((nchw + 1) * CH, 16), jnp.float32),
        pltpu.VMEM_SHARED((NP, 16), jnp.float32),
        pltpu.SemaphoreType.DMA,
    ]
    if with_cnt:
        out_t = (out_t, jax.ShapeDtypeStruct((NC * NP, 16), jnp.float32))
        scratch = scratch[:2] + [
            pltpu.VMEM((CH, 16), jnp.float32),
            pltpu.VMEM_SHARED((NP, 16), jnp.float32),
        ] + scratch[2:]

    def body(idx_hbm, vals_hbm, zeros_hbm, *rest):
        if with_cnt:
            (ones_hbm, out_hbm, cout_hbm, idx_v, vals_v, ones_v, cacc_sh,
             acc_sh, sem) = rest
        else:
            (out_hbm, idx_v, vals_v, acc_sh, sem) = rest
        c = lax.axis_index("c")
        s = lax.axis_index("s")
        wid = s * NC + c
        extra = jnp.where(wid < nxtra, 1, 0)
        crel = wid * nchw + jnp.minimum(wid, nxtra)
        cabs = chunk0 + crel
        nch = nchw + extra
        pltpu.sync_copy(zeros_hbm.at[pl.ds(s * RPS, RPS)],
                        acc_sh.at[pl.ds(s * RPS, RPS)])
        if with_cnt:
            pltpu.sync_copy(zeros_hbm.at[pl.ds(s * RPS, RPS)],
                            cacc_sh.at[pl.ds(s * RPS, RPS)])
            pltpu.sync_copy(ones_hbm, ones_v)
        pltpu.sync_copy(idx_hbm.at[pl.ds(cabs, nchw)],
                        idx_v.at[pl.ds(0, nchw)])
        pltpu.sync_copy(vals_hbm.at[pl.ds(crel * CH, nchw * CH)],
                        vals_v.at[pl.ds(0, nchw * CH)])

        @pl.when(extra == 1)
        def _():
            pltpu.sync_copy(idx_hbm.at[pl.ds(cabs + nchw, 1)],
                            idx_v.at[pl.ds(nchw, 1)])
            pltpu.sync_copy(vals_hbm.at[pl.ds((crel + nchw) * CH, CH)],
                            vals_v.at[pl.ds(nchw * CH, CH)])

        plsc.subcore_barrier()

        def fire(j, carry):
            pltpu.async_copy(vals_v.at[pl.ds(j * CH, CH)],
                             acc_sh.at[idx_v.at[j]], sem, add=True)
            if with_cnt:
                pltpu.async_copy(ones_v, cacc_sh.at[idx_v.at[j]], sem,
                                 add=True)
            return carry

        lax.fori_loop(0, nch, fire, 0)
        n_waits = nch * 2 if with_cnt else nch

        def drain(j, carry):
            pltpu.make_async_copy(vals_v.at[pl.ds(0, CH)],
                                  acc_sh.at[idx_v.at[0]], sem).wait()
            return carry

        lax.fori_loop(0, n_waits, drain, 0)
        plsc.subcore_barrier()
        pltpu.sync_copy(acc_sh.at[pl.ds(s * RPS, RPS)],
                        out_hbm.at[pl.ds(c * NP + s * RPS, RPS)])
        if with_cnt:
            pltpu.sync_copy(cacc_sh.at[pl.ds(s * RPS, RPS)],
                            cout_hbm.at[pl.ds(c * NP + s * RPS, RPS)])

    return functools.partial(
        pl.kernel, mesh=_MESH, compiler_params=_SC_PARAMS,
        out_type=out_t, scratch_types=scratch)(body)


_gather_h = tuple(_make_gather(h * NCHH) for h in range(NHALF))
_scatter_cnt_h = tuple(_make_scatter(h * NCHH, True) for h in range(NHALF))
_scatter_h = tuple(_make_scatter(h * NCHH, False) for h in range(NHALF))


def _unpack(ap):
    return jnp.concatenate([ap[:, 16 * k:16 * (k + 1)] for k in range(8)],
                           axis=0)


def _pack(a):
    r = a.shape[0] // 8
    return jnp.concatenate([a[r * k:r * (k + 1), :] for k in range(8)],
                           axis=1)


def _msg_body(ea_ref, xs_ref, w1_ref, b1_ref, w2_ref, b2_ref, p_ref, q_ref,
              out_ref):
    ea = _unpack(ea_ref[...]).astype(jnp.bfloat16)
    xs = _unpack(xs_ref[...])
    h = jnp.maximum(
        jnp.dot(ea, w1_ref[...].astype(jnp.bfloat16),
                preferred_element_type=jnp.float32)
        + b1_ref[...], 0.0)
    wf = jnp.dot(h.astype(jnp.bfloat16), w2_ref[...].astype(jnp.bfloat16),
                 preferred_element_type=jnp.float32) + b2_ref[...]
    xr = jnp.dot(xs, p_ref[...], preferred_element_type=jnp.float32)
    msg = jnp.dot(xr * wf, q_ref[...], preferred_element_type=jnp.float32)
    out_ref[...] = _pack(msg)


def _edge_messages(half, ea2, xs2, w1, b1, w2, b2, ic, oc):
    grid = EH // BE
    off = half * grid
    ii = jnp.arange(ic * oc) // oc
    oo = jnp.arange(ic * oc) % oc
    p_mat = (ii[None, :] == jnp.arange(ic)[:, None]).astype(jnp.float32)
    q_mat = (oo[:, None] == jnp.arange(oc)[None, :]).astype(jnp.float32)
    b1r = b1.reshape(1, -1)
    b2r = b2.reshape(1, -1)

    def fixed(a):
        return pl.BlockSpec(a.shape, lambda i: (0,) * a.ndim)

    return pl.pallas_call(
        _msg_body,
        grid=(grid,),
        in_specs=[
            pl.BlockSpec((BR, 128), lambda i: (i + off, 0)),
            pl.BlockSpec((BR, 128), lambda i: (i, 0)),
            fixed(w1), fixed(b1r), fixed(w2), fixed(b2r),
            fixed(p_mat), fixed(q_mat),
        ],
        out_specs=pl.BlockSpec((BR, 128), lambda i: (i, 0)),
        out_shape=jax.ShapeDtypeStruct((EH // 8, 128), jnp.float32),
    )(ea2, xs2, w1, b1r, w2, b2r, p_mat, q_mat)


_NR = N // 8
_NPR = NP // 8


def _node_body(x_ref, sa_ref, sb_ref, ca_ref, cb_ref, root_ref, bias_ref,
               out_ref, *, do_relu):
    s = (sa_ref[0:_NR] + sa_ref[_NPR:_NPR + _NR]
         + sb_ref[0:_NR] + sb_ref[_NPR:_NPR + _NR])
    cnt = (ca_ref[0:_NR] + ca_ref[_NPR:_NPR + _NR]
           + cb_ref[0:_NR] + cb_ref[_NPR:_NPR + _NR])
    inv = 1.0 / jnp.maximum(cnt, 1.0)
    x = _unpack(x_ref[...])
    r = jnp.dot(x, root_ref[...], preferred_element_type=jnp.float32)
    o = _pack(r) + s * inv + bias_ref[...]
    if do_relu:
        o = jnp.maximum(o, 0.0)
    out_ref[...] = o


def _node_combine(x2, sa, sb, ca, cb, root, bias, do_relu):
    bias_t = jnp.tile(bias.reshape(1, -1), (1, 8))
    return pl.pallas_call(
        functools.partial(_node_body, do_relu=do_relu),
        out_shape=jax.ShapeDtypeStruct((_NR, 128), jnp.float32),
    )(x2, sa, sb, ca, cb, root, bias_t)


def kernel(x, edge_index, edge_attr, nn1_w1, nn1_b1, nn1_w2, nn1_b2, root1,
           bias1, nn2_w1, nn2_b1, nn2_w2, nn2_b2, root2, bias2):
    src2 = edge_index[0].astype(jnp.int32).reshape(NCHT, CH)
    dst2 = edge_index[1].astype(jnp.int32).reshape(NCHT, CH)
    ea2 = edge_attr.reshape(E // 8, 128)
    x2 = x.reshape(_NR, 128)
    zeros_np = jnp.zeros((NP, 16), jnp.float32)
    ones_ch = jnp.ones((CH, 16), jnp.float32)

    def packp(a):
        return a.reshape(NC * NP // 8, 128)

    xa = _gather_h[0](x, src2)
    xb = _gather_h[1](x, src2)
    m1a = _edge_messages(0, ea2, xa.reshape(EH // 8, 128),
                         nn1_w1, nn1_b1, nn1_w2, nn1_b2, IN, HID)
    s1a, c1a = _scatter_cnt_h[0](dst2, m1a.reshape(EH, 16), zeros_np,
                                 ones_ch)
    m1b = _edge_messages(1, ea2, xb.reshape(EH // 8, 128),
                         nn1_w1, nn1_b1, nn1_w2, nn1_b2, IN, HID)
    s1b, c1b = _scatter_cnt_h[1](dst2, m1b.reshape(EH, 16), zeros_np,
                                 ones_ch)
    h2 = _node_combine(x2, packp(s1a), packp(s1b), packp(c1a), packp(c1b),
                       root1, bias1, True)

    h = h2.reshape(N, 16)
    ha = _gather_h[0](h, src2)
    hb = _gather_h[1](h, src2)
    m2a = _edge_messages(0, ea2, ha.reshape(EH // 8, 128),
                         nn2_w1, nn2_b1, nn2_w2, nn2_b2, HID, OUT)
    s2a = _scatter_h[0](dst2, m2a.reshape(EH, 16), zeros_np)
    m2b = _edge_messages(1, ea2, hb.reshape(EH // 8, 128),
                         nn2_w1, nn2_b1, nn2_w2, nn2_b2, HID, OUT)
    s2b = _scatter_h[1](dst2, m2b.reshape(EH, 16), zeros_np)
    out2 = _node_combine(h2, packp(s2a), packp(s2b), packp(c1a), packp(c1b),
                         root2, bias2, False)
    return out2.reshape(N, 16)
